# baseline (speedup 1.0000x reference)
import functools

import jax
import jax.numpy as jnp
import numpy as np
from jax import lax
from jax.experimental import pallas as pl
from jax.experimental.pallas import tpu as pltpu


def _ru(x, m):
    return (x + m - 1) // m * m


def _body(x_ref, *refs, H, W, WP, K, p, pool, Ho, Wo, Cin, Cout,
          depth, Bblk, SEG, G):
    w_refs = refs[:depth]
    b_ref, p_ref, s_ref, o_ref, col = refs[depth:]
    HWP = H * WP
    OHW = Ho * Wo
    KK = K * K

    wc = lax.broadcasted_iota(jnp.int32, (1, HWP), 1) % WP
    pad_mask = jnp.logical_and(wc >= 1, wc <= W)

    shifts = []
    for t in range(KK):
        kh, kw = t // K, t % K
        shifts.append((kh - p) * WP + (kw - p))

    pool_taps = [t for t in range(KK)
                 if 0 <= t // K - p < pool and 0 <= t % K - p < pool]

    def scatter(col, rows, b, y, tap_set):
        base = b * SEG + G
        for t in tap_set:
            d = shifts[t]
            s0 = base - d
            col[t * rows:(t + 1) * rows, s0:s0 + HWP] = y
            if d < 0:
                col[t * rows:(t + 1) * rows, base:base - d] = \
                    jnp.zeros((rows, -d), y.dtype)
            elif d > 0:
                col[t * rows:(t + 1) * rows, base + HWP - d:base + HWP] = \
                    jnp.zeros((rows, d), y.dtype)

    for b in range(Bblk):
        xb = x_ref[b].astype(jnp.bfloat16)
        xp = jnp.dot(xb, p_ref[...], preferred_element_type=jnp.float32)
        scatter(col, Cin, b, xp.astype(jnp.bfloat16), range(KK))

    for l in range(depth):
        rows = Cin if l == 0 else Cout
        w_l = w_refs[l][...]
        for b in range(Bblk):
            base = b * SEG + G
            acc = jnp.dot(w_l, col[0:KK * rows, base:base + HWP],
                          preferred_element_type=jnp.float32)
            y = jnp.maximum(acc + b_ref[l], 0.0)
            if l < depth - 1:
                y = jnp.where(pad_mask, y, 0.0)
            scatter(col, Cout, b, y.astype(jnp.bfloat16),
                    range(KK) if l < depth - 1 else pool_taps)

    for b in range(Bblk):
        base = b * SEG + G
        m = None
        for t in pool_taps:
            v = col[t * Cout:(t + 1) * Cout, base:base + HWP]
            m = v if m is None else jnp.maximum(m, v)
        pooled = jnp.dot(m, s_ref[...], preferred_element_type=jnp.float32)
        o_ref[b * Cout:(b + 1) * Cout, :] = pooled


def _wperm_body(w_refs, p_refs, o_refs, nw):
    for i in range(nw):
        o_refs[i][...] = jnp.dot(
            w_refs[i][...].astype(jnp.bfloat16), p_refs[min(i, 1)][...],
            preferred_element_type=jnp.float32).astype(jnp.bfloat16)


def _flatten_weights(params, K, Cin, Cout, depth):
    KK = K * K

    def perm(C):
        M = np.zeros((C * KK, C * KK), np.float32)
        for c in range(C):
            for t in range(KK):
                M[c * KK + t, t * C + c] = 1.0
        return jnp.asarray(M, jnp.bfloat16)

    wc = [params[0][0].reshape(Cout, Cin * KK)] + [
        params[l][0].reshape(Cout, Cout * KK) for l in range(1, depth)]
    p0, p12 = perm(Cin), perm(Cout)
    nw = depth

    def body(*refs):
        _wperm_body(refs[:nw], refs[nw:nw + 2], refs[nw + 2:], nw)

    outs = pl.pallas_call(
        body,
        out_shape=[jax.ShapeDtypeStruct(w.shape, jnp.bfloat16) for w in wc],
        in_specs=[pl.BlockSpec(w.shape, lambda *_: (0, 0)) for w in wc]
        + [pl.BlockSpec(p0.shape, lambda *_: (0, 0)),
           pl.BlockSpec(p12.shape, lambda *_: (0, 0))],
        out_specs=[pl.BlockSpec(w.shape, lambda *_: (0, 0)) for w in wc],
        grid=(1,),
    )(*wc, p0, p12)
    return outs


def _place_matrix(H, W, WP):
    P = np.zeros((H * W, H * WP), np.float32)
    for h in range(H):
        for w in range(W):
            P[h * W + w, h * WP + w + 1] = 1.0
    return jnp.asarray(P, jnp.bfloat16)


def _pool_select(H, W, WP, pool):
    Ho, Wo = H // pool, W // pool
    S = np.zeros((H * WP, Ho * Wo), np.float32)
    for oh in range(Ho):
        for ow in range(Wo):
            S[(pool * oh) * WP + pool * ow + 1, oh * Wo + ow] = 1.0
    return jnp.asarray(S, jnp.bfloat16)


def _encoder(img, params, K, pool, batch_blocks):
    B, Cin, H, W = img.shape
    Cout = params[0][0].shape[0]
    depth = len(params)
    p = K // 2
    WP = W + 4
    Ho, Wo = H // pool, W // pool
    HW, HWP, OHW = H * W, H * WP, Ho * Wo
    KK = K * K
    assert B % batch_blocks == 0
    Bblk = B // batch_blocks
    guard = max(p, pool - 1) * (WP + 1)
    G = _ru(guard, 128)
    SEG = G + _ru(HWP + guard, 128)
    Cmax = max(Cin, Cout)

    x = img.reshape(B, Cin, HW)
    wt = _flatten_weights(params, K, Cin, Cout, depth)
    bias = jnp.stack([prm[1].astype(jnp.float32).reshape(Cout, 1)
                      for prm in params])
    place = _place_matrix(H, W, WP)
    sel = _pool_select(H, W, WP, pool)

    out = pl.pallas_call(
        functools.partial(_body, H=H, W=W, WP=WP, K=K, p=p, pool=pool, Ho=Ho,
                          Wo=Wo, Cin=Cin, Cout=Cout, depth=depth, Bblk=Bblk,
                          SEG=SEG, G=G),
        out_shape=jax.ShapeDtypeStruct((B * Cout, OHW), jnp.float32),
        grid=(batch_blocks,),
        in_specs=[pl.BlockSpec((Bblk, Cin, HW), lambda i: (i, 0, 0))]
        + [pl.BlockSpec(w.shape, lambda i: (0, 0)) for w in wt]
        + [
            pl.BlockSpec(bias.shape, lambda i: (0, 0, 0)),
            pl.BlockSpec(place.shape, lambda i: (0, 0)),
            pl.BlockSpec(sel.shape, lambda i: (0, 0)),
        ],
        out_specs=pl.BlockSpec((Bblk * Cout, OHW), lambda i: (i, 0)),
        scratch_shapes=[pltpu.VMEM((KK * Cmax, Bblk * SEG), jnp.bfloat16)],
        compiler_params=pltpu.CompilerParams(
            dimension_semantics=("parallel",)),
    )(x, *wt, bias, place, sel)

    return out.reshape(B, Cout, Ho, Wo)


def kernel(img, w0, b0, w1, b1, w2, b2):
    params = [(w0, b0), (w1, b1), (w2, b2)]
    return _encoder(img, params, 3, 2, batch_blocks=16)


# speedup vs baseline: 1.0478x; 1.0478x over previous
import functools

import jax
import jax.numpy as jnp
import numpy as np
from jax import lax
from jax.experimental import pallas as pl
from jax.experimental.pallas import tpu as pltpu


def _ru(x, m):
    return (x + m - 1) // m * m


def _body(x_ref, w0_ref, w12_ref, b_ref, p_ref, s_ref, o_ref,
          col, *, H, W, WP, K, p, pool, Ho, Wo, Cin, Cout,
          depth, Bblk, SEG, G):
    HWP = H * WP
    OHW = Ho * Wo
    KK = K * K

    wc = lax.broadcasted_iota(jnp.int32, (1, HWP), 1) % WP
    pad_mask = jnp.logical_and(wc >= 1, wc <= W)

    shifts = []
    for t in range(KK):
        kh, kw = t // K, t % K
        shifts.append((kh - p) * WP + (kw - p))

    pool_taps = [t for t in range(KK)
                 if 0 <= t // K - p < pool and 0 <= t % K - p < pool]

    def scatter(col, rows, b, y, tap_set):
        base = b * SEG + G
        for t in tap_set:
            d = shifts[t]
            s0 = base - d
            col[t * rows:(t + 1) * rows, s0:s0 + HWP] = y
            if d < 0:
                col[t * rows:(t + 1) * rows, base:base - d] = \
                    jnp.zeros((rows, -d), y.dtype)
            elif d > 0:
                col[t * rows:(t + 1) * rows, base + HWP - d:base + HWP] = \
                    jnp.zeros((rows, d), y.dtype)

    for b in range(Bblk):
        xb = x_ref[b].astype(jnp.bfloat16)
        xp = jnp.dot(xb, p_ref[...], preferred_element_type=jnp.float32)
        scatter(col, Cin, b, xp.astype(jnp.bfloat16), range(KK))

    for l in range(depth):
        rows = Cin if l == 0 else Cout
        w_l = w0_ref[...] if l == 0 else w12_ref[l - 1]
        for b in range(Bblk):
            base = b * SEG + G
            acc = jnp.dot(w_l, col[0:KK * rows, base:base + HWP],
                          preferred_element_type=jnp.float32)
            y = jnp.maximum(acc + b_ref[l], 0.0)
            if l < depth - 1:
                y = jnp.where(pad_mask, y, 0.0)
            scatter(col, Cout, b, y.astype(jnp.bfloat16),
                    range(KK) if l < depth - 1 else pool_taps)

    for b in range(Bblk):
        base = b * SEG + G
        m = None
        for t in pool_taps:
            v = col[t * Cout:(t + 1) * Cout, base:base + HWP]
            m = v if m is None else jnp.maximum(m, v)
        pooled = jnp.dot(m, s_ref[...], preferred_element_type=jnp.float32)
        o_ref[b * Cout:(b + 1) * Cout, :] = pooled


def _place_matrix(H, W, WP):
    P = np.zeros((H * W, H * WP), np.float32)
    for h in range(H):
        for w in range(W):
            P[h * W + w, h * WP + w + 1] = 1.0
    return jnp.asarray(P, jnp.bfloat16)


def _pool_select(H, W, WP, pool):
    Ho, Wo = H // pool, W // pool
    S = np.zeros((H * WP, Ho * Wo), np.float32)
    for oh in range(Ho):
        for ow in range(Wo):
            S[(pool * oh) * WP + pool * ow + 1, oh * Wo + ow] = 1.0
    return jnp.asarray(S, jnp.bfloat16)


def _encoder(img, params, K, pool, batch_blocks):
    B, Cin, H, W = img.shape
    Cout = params[0][0].shape[0]
    depth = len(params)
    p = K // 2
    WP = W + 4
    Ho, Wo = H // pool, W // pool
    HW, HWP, OHW = H * W, H * WP, Ho * Wo
    KK = K * K
    assert B % batch_blocks == 0
    Bblk = B // batch_blocks
    guard = max(p, pool - 1) * (WP + 1)
    G = _ru(guard, 128)
    SEG = G + _ru(HWP + guard, 128)
    Cmax = max(Cin, Cout)

    x = img.reshape(B, Cin, HW)
    w0 = params[0][0].astype(jnp.bfloat16).transpose(0, 2, 3, 1).reshape(
        Cout, KK * Cin)
    w12 = jnp.stack([
        params[l][0].astype(jnp.bfloat16).transpose(0, 2, 3, 1).reshape(
            Cout, KK * Cout) for l in range(1, depth)])
    bias = jnp.stack([prm[1].astype(jnp.float32).reshape(Cout, 1)
                      for prm in params])
    place = _place_matrix(H, W, WP)
    sel = _pool_select(H, W, WP, pool)

    out = pl.pallas_call(
        functools.partial(_body, H=H, W=W, WP=WP, K=K, p=p, pool=pool, Ho=Ho,
                          Wo=Wo, Cin=Cin, Cout=Cout, depth=depth, Bblk=Bblk,
                          SEG=SEG, G=G),
        out_shape=jax.ShapeDtypeStruct((B * Cout, OHW), jnp.float32),
        grid=(batch_blocks,),
        in_specs=[
            pl.BlockSpec((Bblk, Cin, HW), lambda i: (i, 0, 0)),
            pl.BlockSpec(w0.shape, lambda i: (0, 0)),
            pl.BlockSpec(w12.shape, lambda i: (0, 0, 0)),
            pl.BlockSpec(bias.shape, lambda i: (0, 0, 0)),
            pl.BlockSpec(place.shape, lambda i: (0, 0)),
            pl.BlockSpec(sel.shape, lambda i: (0, 0)),
        ],
        out_specs=pl.BlockSpec((Bblk * Cout, OHW), lambda i: (i, 0)),
        scratch_shapes=[pltpu.VMEM((KK * Cmax, Bblk * SEG), jnp.bfloat16)],
        compiler_params=pltpu.CompilerParams(
            dimension_semantics=("parallel",)),
    )(x, w0, w12, bias, place, sel)

    return out.reshape(B, Cout, Ho, Wo)


def kernel(img, w0, b0, w1, b1, w2, b2):
    params = [(w0, b0), (w1, b1), (w2, b2)]
    return _encoder(img, params, 3, 2, batch_blocks=16)
